# baseline (speedup 1.0000x reference)
import jax
import jax.numpy as jnp
from jax import lax
from jax.experimental import pallas as pl
from jax.experimental.pallas import tpu as pltpu
from jax.experimental.pallas import tpu_sc as plsc

V = 1000000
H = 64
B = 16384
C = 10
K = 5
W = 2 * H

NC = 2
NS = 16
NW = NC * NS
BPW = B // NW
NB = 16
NCHUNK = BPW // NB
NBUF = 2
HV = H // 16
YPC = NB * K // 16

NFB = V // 128
TAILC = V - NFB * 128
NBLK = 2 * NFB
BPWK = (NBLK + NW - 1) // NW


def _relayout_body(e1t_hbm, e2t_hbm, tail1_hbm, tail2_hbm, e1p_hbm, e2p_hbm,
                   ibufs, obufs, in_sems, out_sems):
    wid = lax.axis_index("s") * NC + lax.axis_index("c")
    iota = lax.broadcasted_iota(jnp.int32, (16,), 0)
    rows_d = [iota + 16 * d for d in range(HV)]

    def start_in(g, b):
        t = g // NFB
        j = g - t * NFB
        off = j * 128

        @pl.when(t == 0)
        def _():
            pltpu.make_async_copy(e1t_hbm.at[:, pl.ds(off, 128)], ibufs[b],
                                  in_sems[b]).start()

        @pl.when(t != 0)
        def _():
            pltpu.make_async_copy(e2t_hbm.at[:, pl.ds(off, 128)], ibufs[b],
                                  in_sems[b]).start()

    def wait_in(b):
        pltpu.make_async_copy(e1t_hbm.at[:, pl.ds(0, 128)], ibufs[b],
                              in_sems[b]).wait()

    def start_out(g, b):
        t = g // NFB
        j = g - t * NFB
        off = j * 64

        @pl.when(t == 0)
        def _():
            pltpu.make_async_copy(obufs[b], e1p_hbm.at[pl.ds(off, 64), :],
                                  out_sems[b]).start()

        @pl.when(t != 0)
        def _():
            pltpu.make_async_copy(obufs[b], e2p_hbm.at[pl.ds(off, 64), :],
                                  out_sems[b]).start()

    def wait_out(b):
        pltpu.make_async_copy(obufs[b], e1p_hbm.at[pl.ds(0, 64), :],
                              out_sems[b]).wait()

    def transpose_block(b, nrows):
        ib = ibufs[b]
        ob = obufs[b]
        for r in range(nrows):
            for p in range(2):
                cols = jnp.full((16,), 2 * r + p, jnp.int32)
                for d in range(HV):
                    ob[r, pl.ds(p * H + d * 16, 16)] = plsc.load_gather(
                        ib, [rows_d[d], cols])

    start_in(wid, 0)
    NI2 = (BPWK + NBUF - 1) // NBUF

    def loop_body(i2, carry):
        for b in range(NBUF):
            i = i2 * NBUF + b
            g = wid + i * NW

            @pl.when(g < NBLK)
            def _():
                gn = g + NW

                @pl.when(gn < NBLK)
                def _():
                    start_in(gn, (b + 1) % NBUF)

                @pl.when(i >= NBUF)
                def _():
                    wait_out(b)

                wait_in(b)
                transpose_block(b, 64)
                start_out(g, b)
        return carry

    lax.fori_loop(0, NI2, loop_body, 0)

    for b in range(NBUF):
        wait_out(b)

    @pl.when(wid < 2)
    def _():
        @pl.when(wid == 0)
        def _():
            pltpu.make_async_copy(tail1_hbm, obufs[0].at[pl.ds(0, TAILC // 2), :],
                                  in_sems[0]).start()

        @pl.when(wid == 1)
        def _():
            pltpu.make_async_copy(tail2_hbm, obufs[0].at[pl.ds(0, TAILC // 2), :],
                                  in_sems[0]).start()

        pltpu.make_async_copy(tail1_hbm, obufs[0].at[pl.ds(0, TAILC // 2), :],
                              in_sems[0]).wait()

        @pl.when(wid == 0)
        def _():
            pltpu.make_async_copy(obufs[0].at[pl.ds(0, TAILC // 2), :],
                                  e1p_hbm.at[pl.ds(NFB * 64, TAILC // 2), :],
                                  out_sems[0]).start()

        @pl.when(wid == 1)
        def _():
            pltpu.make_async_copy(obufs[0].at[pl.ds(0, TAILC // 2), :],
                                  e2p_hbm.at[pl.ds(NFB * 64, TAILC // 2), :],
                                  out_sems[0]).start()

        pltpu.make_async_copy(obufs[0].at[pl.ds(0, TAILC // 2), :],
                              e1p_hbm.at[pl.ds(0, TAILC // 2), :],
                              out_sems[0]).wait()


def _cbow_body(ctx_hbm, cph_hbm, cen_hbm, knp_hbm, e1_hbm, e2_hbm, out_hbm,
               ctx_v, cph_v, cen_v, knp_v, ybuf, e1_bufs, e2_bufs,
               idx_sem, e1_sems, e2_sems, out_sem):
    wid = lax.axis_index("s") * NC + lax.axis_index("c")
    ctx_base = pl.multiple_of(wid * (BPW * C), 8)
    cen_base = pl.multiple_of(wid * (BPW * K), 8)

    for src, n, dst in ((ctx_hbm, BPW * C, ctx_v), (cph_hbm, BPW * C, cph_v),
                        (cen_hbm, BPW * K, cen_v), (knp_hbm, BPW * K, knp_v)):
        base = ctx_base if n == BPW * C else cen_base
        pltpu.make_async_copy(src.at[pl.ds(base, n)], dst, idx_sem).start()
    for src, n, dst in ((ctx_hbm, BPW * C, ctx_v), (cph_hbm, BPW * C, cph_v),
                        (cen_hbm, BPW * K, cen_v), (knp_hbm, BPW * K, knp_v)):
        base = ctx_base if n == BPW * C else cen_base
        pltpu.make_async_copy(src.at[pl.ds(base, n)], dst, idx_sem).wait()

    lanes = lax.broadcasted_iota(jnp.int32, (16,), 0)
    masks = [lanes == l for l in range(16)]

    def start_gather(j, b):
        joff_c = pl.multiple_of(j * (NB * C), 8)
        joff_k = pl.multiple_of(j * (NB * K), 8)
        half = NB * C // 2
        pltpu.make_async_copy(e1_hbm.at[ctx_v.at[pl.ds(joff_c, half)]],
                              e1_bufs[b].at[pl.ds(0, half)],
                              e1_sems[b]).start()
        pltpu.make_async_copy(e1_hbm.at[ctx_v.at[pl.ds(joff_c + half, half)]],
                              e1_bufs[b].at[pl.ds(half, half)],
                              e1_sems[b]).start()
        pltpu.make_async_copy(e2_hbm.at[cen_v.at[pl.ds(joff_k, NB * K)]],
                              e2_bufs[b], e2_sems[b]).start()

    def wait_gather(b):
        half = NB * C // 2
        pltpu.make_async_copy(e1_hbm.at[ctx_v.at[pl.ds(0, half)]],
                              e1_bufs[b].at[pl.ds(0, half)],
                              e1_sems[b]).wait()
        pltpu.make_async_copy(e1_hbm.at[ctx_v.at[pl.ds(0, half)]],
                              e1_bufs[b].at[pl.ds(half, half)],
                              e1_sems[b]).wait()
        pltpu.make_async_copy(e2_hbm.at[cen_v.at[pl.ds(0, NB * K)]],
                              e2_bufs[b], e2_sems[b]).wait()

    for b in range(NBUF):
        start_gather(b, b)

    def chunk_compute(j, b):
        wait_gather(b)
        e1b = e1_bufs[b]
        e2b = e2_bufs[b]
        joff_c = pl.multiple_of(j * (NB * C), 8)
        joff_k = pl.multiple_of(j * (NB * K), 8)
        cph_w = [cph_v[pl.ds(joff_c + w * 16, 16)] for w in range(NB * C // 16)]
        knp_w = [knp_v[pl.ds(joff_k + w * 16, 16)] for w in range(NB * K // 16)]
        accs = [jnp.zeros((16,), jnp.float32) for _ in range(YPC)]
        for r in range(NB):
            def coff(e):
                return cph_w[e // 16][e % 16]
            off = coff(r * C)
            hacc = [e1b[r * C, pl.ds(off + d * 16, 16)] for d in range(HV)]
            for c in range(1, C):
                off = coff(r * C + c)
                for d in range(HV):
                    hacc[d] = hacc[d] + e1b[r * C + c, pl.ds(off + d * 16, 16)]
            h = [a * (1.0 / C) for a in hacc]
            for k in range(K):
                q = r * K + k
                koff = knp_w[q // 16][q % 16]
                p0 = e2b[q, pl.ds(koff, 16)] * h[0]
                p1 = e2b[q, pl.ds(koff + 16, 16)] * h[1]
                p2 = e2b[q, pl.ds(koff + 32, 16)] * h[2]
                p3 = e2b[q, pl.ds(koff + 48, 16)] * h[3]
                s = jnp.sum((p0 + p1) + (p2 + p3))
                accs[q // 16] = jnp.where(masks[q % 16], s, accs[q // 16])
        ybase = j * (NB * K)
        for v in range(YPC):
            y = 1.0 / (1.0 + jnp.exp(-accs[v]))
            ybuf[pl.ds(pl.multiple_of(ybase + v * 16, 8), 16)] = y

    def loop_body(g, carry):
        for b in range(NBUF):
            j = g * NBUF + b
            chunk_compute(j, b)

            @pl.when(j + NBUF < NCHUNK)
            def _():
                start_gather(j + NBUF, b)
        return carry

    lax.fori_loop(0, NCHUNK // NBUF, loop_body, 0)

    out_base = pl.multiple_of(wid * (BPW * K), 8)
    pltpu.make_async_copy(ybuf, out_hbm.at[pl.ds(out_base, BPW * K)],
                          out_sem).start()
    pltpu.make_async_copy(ybuf, out_hbm.at[pl.ds(out_base, BPW * K)],
                          out_sem).wait()


@jax.jit
def _cbow_sc(ctx_pair, ctx_phase, cen_pair, cen_phase, E1t, E2t,
             tail1, tail2):
    mesh = plsc.VectorSubcoreMesh(core_axis_name="c", subcore_axis_name="s",
                                  num_cores=NC, num_subcores=NS)
    relayout = pl.kernel(
        _relayout_body,
        out_type=(jax.ShapeDtypeStruct((V // 2, W), jnp.float32),
                  jax.ShapeDtypeStruct((V // 2, W), jnp.float32)),
        mesh=mesh,
        compiler_params=pltpu.CompilerParams(needs_layout_passes=False),
        scratch_types=[
            [pltpu.VMEM((H, 128), jnp.float32) for _ in range(NBUF)],
            [pltpu.VMEM((H, W), jnp.float32) for _ in range(NBUF)],
            [pltpu.SemaphoreType.DMA for _ in range(NBUF)],
            [pltpu.SemaphoreType.DMA for _ in range(NBUF)],
        ],
    )
    E1p, E2p = relayout(E1t, E2t, tail1, tail2)

    kern = pl.kernel(
        _cbow_body,
        out_type=jax.ShapeDtypeStruct((B * K,), jnp.float32),
        mesh=mesh,
        compiler_params=pltpu.CompilerParams(needs_layout_passes=False),
        scratch_types=[
            pltpu.VMEM((BPW * C,), jnp.int32),
            pltpu.VMEM((BPW * C,), jnp.int32),
            pltpu.VMEM((BPW * K,), jnp.int32),
            pltpu.VMEM((BPW * K,), jnp.int32),
            pltpu.VMEM((BPW * K,), jnp.float32),
            [pltpu.VMEM((NB * C, W), jnp.float32) for _ in range(NBUF)],
            [pltpu.VMEM((NB * K, W), jnp.float32) for _ in range(NBUF)],
            pltpu.SemaphoreType.DMA,
            [pltpu.SemaphoreType.DMA for _ in range(NBUF)],
            [pltpu.SemaphoreType.DMA for _ in range(NBUF)],
            pltpu.SemaphoreType.DMA,
        ],
    )
    return kern(ctx_pair, ctx_phase, cen_pair, cen_phase, E1p, E2p)


def kernel(contexts, centers, E1, E2):
    ctx_flat = contexts.reshape(B * C).astype(jnp.int32)
    cen_flat = centers.reshape(B * K).astype(jnp.int32)
    tail1 = E1[NFB * 128:].reshape(TAILC // 2, W)
    tail2 = E2[NFB * 128:].reshape(TAILC // 2, W)
    y = _cbow_sc(ctx_flat >> 1, (ctx_flat & 1) * H,
                 cen_flat >> 1, (cen_flat & 1) * H,
                 E1.T, E2.T, tail1, tail2)
    return y.reshape(B, K)


# speedup vs baseline: 1.1614x; 1.1614x over previous
import jax
import jax.numpy as jnp
from jax import lax
from jax.experimental import pallas as pl
from jax.experimental.pallas import tpu as pltpu
from jax.experimental.pallas import tpu_sc as plsc

V = 1000000
H = 64
B = 16384
C = 10
K = 5
W = 2 * H

NC = 2
NS = 16
NW = NC * NS
BPW = B // NW
NB = 16
NCHUNK = BPW // NB
NBUF = 2
HV = H // 16
YPC = NB * K // 16

NFB = V // 128
TAILC = V - NFB * 128
NBLK = 2 * NFB
BPWK = (NBLK + NW - 1) // NW


def _relayout_body(e1t_hbm, e2t_hbm, tail1_hbm, tail2_hbm, e1p_hbm, e2p_hbm,
                   ibufs, obufs, in_sems, out_sems):
    wid = lax.axis_index("s") * NC + lax.axis_index("c")
    iota = lax.broadcasted_iota(jnp.int32, (16,), 0)
    rows_d = [iota + 16 * d for d in range(HV)]

    def start_in(g, b):
        t = g // NFB
        j = g - t * NFB
        off = j * 128

        @pl.when(t == 0)
        def _():
            pltpu.make_async_copy(e1t_hbm.at[:, pl.ds(off, 128)],
                                  ibufs[b].at[:, pl.ds(0, 128)],
                                  in_sems[b]).start()

        @pl.when(t != 0)
        def _():
            pltpu.make_async_copy(e2t_hbm.at[:, pl.ds(off, 128)],
                                  ibufs[b].at[:, pl.ds(0, 128)],
                                  in_sems[b]).start()

    def wait_in(b):
        pltpu.make_async_copy(e1t_hbm.at[:, pl.ds(0, 128)],
                              ibufs[b].at[:, pl.ds(0, 128)],
                              in_sems[b]).wait()

    def start_out(g, b):
        t = g // NFB
        j = g - t * NFB
        off = j * 64

        @pl.when(t == 0)
        def _():
            pltpu.make_async_copy(obufs[b], e1p_hbm.at[pl.ds(off, 64), :],
                                  out_sems[b]).start()

        @pl.when(t != 0)
        def _():
            pltpu.make_async_copy(obufs[b], e2p_hbm.at[pl.ds(off, 64), :],
                                  out_sems[b]).start()

    def wait_out(b):
        pltpu.make_async_copy(obufs[b], e1p_hbm.at[pl.ds(0, 64), :],
                              out_sems[b]).wait()

    def transpose_block(b, nrows):
        ib = ibufs[b]
        ob = obufs[b]

        def row_body(r, carry):
            for p in range(2):
                cols = jnp.zeros((16,), jnp.int32) + (2 * r + p)
                for d in range(HV):
                    ob[r, pl.ds(p * H + d * 16, 16)] = plsc.load_gather(
                        ib, [rows_d[d], cols])
            return carry

        lax.fori_loop(0, nrows, row_body, 0)

    start_in(wid, 0)
    NI2 = (BPWK + NBUF - 1) // NBUF

    def loop_body(i2, carry):
        for b in range(NBUF):
            i = i2 * NBUF + b
            g = wid + i * NW

            @pl.when(g < NBLK)
            def _():
                gn = g + NW

                @pl.when(gn < NBLK)
                def _():
                    start_in(gn, (b + 1) % NBUF)

                @pl.when(i >= NBUF)
                def _():
                    wait_out(b)

                wait_in(b)
                transpose_block(b, 64)
                start_out(g, b)
        return carry

    lax.fori_loop(0, NI2, loop_body, 0)

    for b in range(NBUF):
        wait_out(b)

    @pl.when(wid < 2)
    def _():
        @pl.when(wid == 0)
        def _():
            pltpu.make_async_copy(tail1_hbm, obufs[0].at[pl.ds(0, TAILC // 2), :],
                                  in_sems[0]).start()

        @pl.when(wid == 1)
        def _():
            pltpu.make_async_copy(tail2_hbm, obufs[0].at[pl.ds(0, TAILC // 2), :],
                                  in_sems[0]).start()

        pltpu.make_async_copy(tail1_hbm, obufs[0].at[pl.ds(0, TAILC // 2), :],
                              in_sems[0]).wait()

        @pl.when(wid == 0)
        def _():
            pltpu.make_async_copy(obufs[0].at[pl.ds(0, TAILC // 2), :],
                                  e1p_hbm.at[pl.ds(NFB * 64, TAILC // 2), :],
                                  out_sems[0]).start()

        @pl.when(wid == 1)
        def _():
            pltpu.make_async_copy(obufs[0].at[pl.ds(0, TAILC // 2), :],
                                  e2p_hbm.at[pl.ds(NFB * 64, TAILC // 2), :],
                                  out_sems[0]).start()

        pltpu.make_async_copy(obufs[0].at[pl.ds(0, TAILC // 2), :],
                              e1p_hbm.at[pl.ds(0, TAILC // 2), :],
                              out_sems[0]).wait()


def _cbow_body(ctx_hbm, cph_hbm, cen_hbm, knp_hbm, e1_hbm, e2_hbm, out_hbm,
               ctx_v, cph_v, cen_v, knp_v, ybuf, e1_bufs, e2_bufs,
               idx_sem, e1_sems, e2_sems, out_sem):
    wid = lax.axis_index("s") * NC + lax.axis_index("c")
    ctx_base = pl.multiple_of(wid * (BPW * C), 8)
    cen_base = pl.multiple_of(wid * (BPW * K), 8)

    for src, n, dst in ((ctx_hbm, BPW * C, ctx_v), (cph_hbm, BPW * C, cph_v),
                        (cen_hbm, BPW * K, cen_v), (knp_hbm, BPW * K, knp_v)):
        base = ctx_base if n == BPW * C else cen_base
        pltpu.make_async_copy(src.at[pl.ds(base, n)], dst, idx_sem).start()
    for src, n, dst in ((ctx_hbm, BPW * C, ctx_v), (cph_hbm, BPW * C, cph_v),
                        (cen_hbm, BPW * K, cen_v), (knp_hbm, BPW * K, knp_v)):
        base = ctx_base if n == BPW * C else cen_base
        pltpu.make_async_copy(src.at[pl.ds(base, n)], dst, idx_sem).wait()

    lanes = lax.broadcasted_iota(jnp.int32, (16,), 0)
    masks = [lanes == l for l in range(16)]

    def start_gather(j, b):
        joff_c = pl.multiple_of(j * (NB * C), 8)
        joff_k = pl.multiple_of(j * (NB * K), 8)
        half = NB * C // 2
        pltpu.make_async_copy(e1_hbm.at[ctx_v.at[pl.ds(joff_c, half)]],
                              e1_bufs[b].at[pl.ds(0, half)],
                              e1_sems[b]).start()
        pltpu.make_async_copy(e1_hbm.at[ctx_v.at[pl.ds(joff_c + half, half)]],
                              e1_bufs[b].at[pl.ds(half, half)],
                              e1_sems[b]).start()
        pltpu.make_async_copy(e2_hbm.at[cen_v.at[pl.ds(joff_k, NB * K)]],
                              e2_bufs[b], e2_sems[b]).start()

    def wait_gather(b):
        half = NB * C // 2
        pltpu.make_async_copy(e1_hbm.at[ctx_v.at[pl.ds(0, half)]],
                              e1_bufs[b].at[pl.ds(0, half)],
                              e1_sems[b]).wait()
        pltpu.make_async_copy(e1_hbm.at[ctx_v.at[pl.ds(0, half)]],
                              e1_bufs[b].at[pl.ds(half, half)],
                              e1_sems[b]).wait()
        pltpu.make_async_copy(e2_hbm.at[cen_v.at[pl.ds(0, NB * K)]],
                              e2_bufs[b], e2_sems[b]).wait()

    for b in range(NBUF):
        start_gather(b, b)

    def chunk_compute(j, b):
        wait_gather(b)
        e1b = e1_bufs[b]
        e2b = e2_bufs[b]
        joff_c = pl.multiple_of(j * (NB * C), 8)
        joff_k = pl.multiple_of(j * (NB * K), 8)
        cph_w = [cph_v[pl.ds(joff_c + w * 16, 16)] for w in range(NB * C // 16)]
        knp_w = [knp_v[pl.ds(joff_k + w * 16, 16)] for w in range(NB * K // 16)]
        accs = [jnp.zeros((16,), jnp.float32) for _ in range(YPC)]
        for r in range(NB):
            def coff(e):
                return cph_w[e // 16][e % 16]
            off = coff(r * C)
            hacc = [e1b[r * C, pl.ds(off + d * 16, 16)] for d in range(HV)]
            for c in range(1, C):
                off = coff(r * C + c)
                for d in range(HV):
                    hacc[d] = hacc[d] + e1b[r * C + c, pl.ds(off + d * 16, 16)]
            h = [a * (1.0 / C) for a in hacc]
            for k in range(K):
                q = r * K + k
                koff = knp_w[q // 16][q % 16]
                p0 = e2b[q, pl.ds(koff, 16)] * h[0]
                p1 = e2b[q, pl.ds(koff + 16, 16)] * h[1]
                p2 = e2b[q, pl.ds(koff + 32, 16)] * h[2]
                p3 = e2b[q, pl.ds(koff + 48, 16)] * h[3]
                s = jnp.sum((p0 + p1) + (p2 + p3))
                accs[q // 16] = jnp.where(masks[q % 16], s, accs[q // 16])
        ybase = j * (NB * K)
        for v in range(YPC):
            y = 1.0 / (1.0 + jnp.exp(-accs[v]))
            ybuf[pl.ds(pl.multiple_of(ybase + v * 16, 8), 16)] = y

    def loop_body(g, carry):
        for b in range(NBUF):
            j = g * NBUF + b
            chunk_compute(j, b)

            @pl.when(j + NBUF < NCHUNK)
            def _():
                start_gather(j + NBUF, b)
        return carry

    lax.fori_loop(0, NCHUNK // NBUF, loop_body, 0)

    out_base = pl.multiple_of(wid * (BPW * K), 8)
    pltpu.make_async_copy(ybuf, out_hbm.at[pl.ds(out_base, BPW * K)],
                          out_sem).start()
    pltpu.make_async_copy(ybuf, out_hbm.at[pl.ds(out_base, BPW * K)],
                          out_sem).wait()


@jax.jit
def _cbow_sc(ctx_pair, ctx_phase, cen_pair, cen_phase, E1t, E2t,
             tail1, tail2):
    mesh = plsc.VectorSubcoreMesh(core_axis_name="c", subcore_axis_name="s",
                                  num_cores=NC, num_subcores=NS)
    relayout = pl.kernel(
        _relayout_body,
        out_type=(jax.ShapeDtypeStruct((V // 2, W), jnp.float32),
                  jax.ShapeDtypeStruct((V // 2, W), jnp.float32)),
        mesh=mesh,
        compiler_params=pltpu.CompilerParams(needs_layout_passes=False),
        scratch_types=[
            [pltpu.VMEM((H, 136), jnp.float32) for _ in range(NBUF)],
            [pltpu.VMEM((H, W), jnp.float32) for _ in range(NBUF)],
            [pltpu.SemaphoreType.DMA for _ in range(NBUF)],
            [pltpu.SemaphoreType.DMA for _ in range(NBUF)],
        ],
    )
    E1p, E2p = relayout(E1t, E2t, tail1, tail2)

    kern = pl.kernel(
        _cbow_body,
        out_type=jax.ShapeDtypeStruct((B * K,), jnp.float32),
        mesh=mesh,
        compiler_params=pltpu.CompilerParams(needs_layout_passes=False),
        scratch_types=[
            pltpu.VMEM((BPW * C,), jnp.int32),
            pltpu.VMEM((BPW * C,), jnp.int32),
            pltpu.VMEM((BPW * K,), jnp.int32),
            pltpu.VMEM((BPW * K,), jnp.int32),
            pltpu.VMEM((BPW * K,), jnp.float32),
            [pltpu.VMEM((NB * C, W), jnp.float32) for _ in range(NBUF)],
            [pltpu.VMEM((NB * K, W), jnp.float32) for _ in range(NBUF)],
            pltpu.SemaphoreType.DMA,
            [pltpu.SemaphoreType.DMA for _ in range(NBUF)],
            [pltpu.SemaphoreType.DMA for _ in range(NBUF)],
            pltpu.SemaphoreType.DMA,
        ],
    )
    return kern(ctx_pair, ctx_phase, cen_pair, cen_phase, E1p, E2p)


def kernel(contexts, centers, E1, E2):
    ctx_flat = contexts.reshape(B * C).astype(jnp.int32)
    cen_flat = centers.reshape(B * K).astype(jnp.int32)
    tail1 = E1[NFB * 128:].reshape(TAILC // 2, W)
    tail2 = E2[NFB * 128:].reshape(TAILC // 2, W)
    y = _cbow_sc(ctx_flat >> 1, (ctx_flat & 1) * H,
                 cen_flat >> 1, (cen_flat & 1) * H,
                 E1.T, E2.T, tail1, tail2)
    return y.reshape(B, K)


# speedup vs baseline: 2.9719x; 2.5588x over previous
import jax
import jax.numpy as jnp
from jax import lax
from jax.experimental import pallas as pl
from jax.experimental.pallas import tpu as pltpu
from jax.experimental.pallas import tpu_sc as plsc

V = 1000000
H = 64
B = 16384
C = 10
K = 5
W = 2 * H

NC = 2
NS = 16
NW = NC * NS
BPW = B // NW
NB = 16
NCHUNK = BPW // NB
NBUF = 2
HV = H // 16
YPC = NB * K // 16

def _cbow_body(ctx_hbm, cph_hbm, cen_hbm, knp_hbm, e1_hbm, e2_hbm, out_hbm,
               ctx_v, cph_v, cen_v, knp_v, ybuf, e1_bufs, e2_bufs,
               idx_sem, e1_sems, e2_sems, out_sem):
    wid = lax.axis_index("s") * NC + lax.axis_index("c")
    ctx_base = pl.multiple_of(wid * (BPW * C), 8)
    cen_base = pl.multiple_of(wid * (BPW * K), 8)

    for src, n, dst in ((ctx_hbm, BPW * C, ctx_v), (cph_hbm, BPW * C, cph_v),
                        (cen_hbm, BPW * K, cen_v), (knp_hbm, BPW * K, knp_v)):
        base = ctx_base if n == BPW * C else cen_base
        pltpu.make_async_copy(src.at[pl.ds(base, n)], dst, idx_sem).start()
    for src, n, dst in ((ctx_hbm, BPW * C, ctx_v), (cph_hbm, BPW * C, cph_v),
                        (cen_hbm, BPW * K, cen_v), (knp_hbm, BPW * K, knp_v)):
        base = ctx_base if n == BPW * C else cen_base
        pltpu.make_async_copy(src.at[pl.ds(base, n)], dst, idx_sem).wait()

    lanes = lax.broadcasted_iota(jnp.int32, (16,), 0)
    masks = [lanes == l for l in range(16)]

    def start_gather(j, b):
        joff_c = pl.multiple_of(j * (NB * C), 8)
        joff_k = pl.multiple_of(j * (NB * K), 8)
        half = NB * C // 2
        pltpu.make_async_copy(e1_hbm.at[ctx_v.at[pl.ds(joff_c, half)]],
                              e1_bufs[b].at[pl.ds(0, half)],
                              e1_sems[b]).start()
        pltpu.make_async_copy(e1_hbm.at[ctx_v.at[pl.ds(joff_c + half, half)]],
                              e1_bufs[b].at[pl.ds(half, half)],
                              e1_sems[b]).start()
        pltpu.make_async_copy(e2_hbm.at[cen_v.at[pl.ds(joff_k, NB * K)]],
                              e2_bufs[b], e2_sems[b]).start()

    def wait_gather(b):
        half = NB * C // 2
        pltpu.make_async_copy(e1_hbm.at[ctx_v.at[pl.ds(0, half)]],
                              e1_bufs[b].at[pl.ds(0, half)],
                              e1_sems[b]).wait()
        pltpu.make_async_copy(e1_hbm.at[ctx_v.at[pl.ds(0, half)]],
                              e1_bufs[b].at[pl.ds(half, half)],
                              e1_sems[b]).wait()
        pltpu.make_async_copy(e2_hbm.at[cen_v.at[pl.ds(0, NB * K)]],
                              e2_bufs[b], e2_sems[b]).wait()

    for b in range(NBUF):
        start_gather(b, b)

    def chunk_compute(j, b):
        wait_gather(b)
        e1b = e1_bufs[b]
        e2b = e2_bufs[b]
        joff_c = pl.multiple_of(j * (NB * C), 8)
        joff_k = pl.multiple_of(j * (NB * K), 8)
        cph_w = [cph_v[pl.ds(joff_c + w * 16, 16)] for w in range(NB * C // 16)]
        knp_w = [knp_v[pl.ds(joff_k + w * 16, 16)] for w in range(NB * K // 16)]
        accs = [jnp.zeros((16,), jnp.float32) for _ in range(YPC)]
        for r in range(NB):
            def coff(e):
                return cph_w[e // 16][e % 16]
            off = coff(r * C)
            hacc = [e1b[r * C, pl.ds(off + d * 16, 16)] for d in range(HV)]
            for c in range(1, C):
                off = coff(r * C + c)
                for d in range(HV):
                    hacc[d] = hacc[d] + e1b[r * C + c, pl.ds(off + d * 16, 16)]
            h = [a * (1.0 / C) for a in hacc]
            for k in range(K):
                q = r * K + k
                koff = knp_w[q // 16][q % 16]
                p0 = e2b[q, pl.ds(koff, 16)] * h[0]
                p1 = e2b[q, pl.ds(koff + 16, 16)] * h[1]
                p2 = e2b[q, pl.ds(koff + 32, 16)] * h[2]
                p3 = e2b[q, pl.ds(koff + 48, 16)] * h[3]
                s = jnp.sum((p0 + p1) + (p2 + p3))
                accs[q // 16] = jnp.where(masks[q % 16], s, accs[q // 16])
        ybase = j * (NB * K)
        for v in range(YPC):
            y = 1.0 / (1.0 + jnp.exp(-accs[v]))
            ybuf[pl.ds(pl.multiple_of(ybase + v * 16, 8), 16)] = y

    def loop_body(g, carry):
        for b in range(NBUF):
            j = g * NBUF + b
            chunk_compute(j, b)

            @pl.when(j + NBUF < NCHUNK)
            def _():
                start_gather(j + NBUF, b)
        return carry

    lax.fori_loop(0, NCHUNK // NBUF, loop_body, 0)

    out_base = pl.multiple_of(wid * (BPW * K), 8)
    pltpu.make_async_copy(ybuf, out_hbm.at[pl.ds(out_base, BPW * K)],
                          out_sem).start()
    pltpu.make_async_copy(ybuf, out_hbm.at[pl.ds(out_base, BPW * K)],
                          out_sem).wait()


@jax.jit
def _cbow_sc(ctx_pair, ctx_phase, cen_pair, cen_phase, E1p, E2p):
    mesh = plsc.VectorSubcoreMesh(core_axis_name="c", subcore_axis_name="s",
                                  num_cores=NC, num_subcores=NS)
    kern = pl.kernel(
        _cbow_body,
        out_type=jax.ShapeDtypeStruct((B * K,), jnp.float32),
        mesh=mesh,
        compiler_params=pltpu.CompilerParams(needs_layout_passes=False),
        scratch_types=[
            pltpu.VMEM((BPW * C,), jnp.int32),
            pltpu.VMEM((BPW * C,), jnp.int32),
            pltpu.VMEM((BPW * K,), jnp.int32),
            pltpu.VMEM((BPW * K,), jnp.int32),
            pltpu.VMEM((BPW * K,), jnp.float32),
            [pltpu.VMEM((NB * C, W), jnp.float32) for _ in range(NBUF)],
            [pltpu.VMEM((NB * K, W), jnp.float32) for _ in range(NBUF)],
            pltpu.SemaphoreType.DMA,
            [pltpu.SemaphoreType.DMA for _ in range(NBUF)],
            [pltpu.SemaphoreType.DMA for _ in range(NBUF)],
            pltpu.SemaphoreType.DMA,
        ],
    )
    return kern(ctx_pair, ctx_phase, cen_pair, cen_phase, E1p, E2p)


def kernel(contexts, centers, E1, E2):
    ctx_flat = contexts.reshape(B * C).astype(jnp.int32)
    cen_flat = centers.reshape(B * K).astype(jnp.int32)
    y = _cbow_sc(ctx_flat >> 1, (ctx_flat & 1) * H,
                 cen_flat >> 1, (cen_flat & 1) * H,
                 E1.reshape(V // 2, W), E2.reshape(V // 2, W))
    return y.reshape(B, K)


# speedup vs baseline: 3.5407x; 1.1914x over previous
import jax
import jax.numpy as jnp
from jax import lax
from jax.experimental import pallas as pl
from jax.experimental.pallas import tpu as pltpu
from jax.experimental.pallas import tpu_sc as plsc

V = 1000000
H = 64
B = 16384
C = 10
K = 5
W = 2 * H

NC = 2
NS = 16
NW = NC * NS
BPW = B // NW
NB = 16
NCHUNK = BPW // NB
NBUF = 2
HV = H // 16
YPC = NB * K // 16


def _cbow_body(ctx_hbm, cen_hbm, tab_hbm, out_hbm,
               ctx_v, cen_v, ybuf, e1_bufs, e2_bufs,
               idx_sem, e1_sems, e2_sems, out_sem):
    wid = lax.axis_index("s") * NC + lax.axis_index("c")
    ctx_base = pl.multiple_of(wid * (BPW * C), 8)
    cen_base = pl.multiple_of(wid * (BPW * K), 8)

    pltpu.make_async_copy(ctx_hbm.at[pl.ds(ctx_base, BPW * C)], ctx_v,
                          idx_sem).start()
    pltpu.make_async_copy(cen_hbm.at[pl.ds(cen_base, BPW * K)], cen_v,
                          idx_sem).start()
    pltpu.make_async_copy(ctx_hbm.at[pl.ds(ctx_base, BPW * C)], ctx_v,
                          idx_sem).wait()
    pltpu.make_async_copy(cen_hbm.at[pl.ds(cen_base, BPW * K)], cen_v,
                          idx_sem).wait()

    lanes = lax.broadcasted_iota(jnp.int32, (16,), 0)
    masks = [lanes == l for l in range(16)]

    def start_gather(j, b):
        joff_c = pl.multiple_of(j * (NB * C), 8)
        joff_k = pl.multiple_of(j * (NB * K), 8)
        half = NB * C // 2
        pltpu.make_async_copy(tab_hbm.at[ctx_v.at[pl.ds(joff_c, half)]],
                              e1_bufs[b].at[pl.ds(0, half)],
                              e1_sems[b]).start()
        pltpu.make_async_copy(tab_hbm.at[ctx_v.at[pl.ds(joff_c + half, half)]],
                              e1_bufs[b].at[pl.ds(half, half)],
                              e1_sems[b]).start()
        pltpu.make_async_copy(tab_hbm.at[cen_v.at[pl.ds(joff_k, NB * K)]],
                              e2_bufs[b], e2_sems[b]).start()

    def wait_gather(b):
        half = NB * C // 2
        pltpu.make_async_copy(tab_hbm.at[ctx_v.at[pl.ds(0, half)]],
                              e1_bufs[b].at[pl.ds(0, half)],
                              e1_sems[b]).wait()
        pltpu.make_async_copy(tab_hbm.at[ctx_v.at[pl.ds(0, half)]],
                              e1_bufs[b].at[pl.ds(half, half)],
                              e1_sems[b]).wait()
        pltpu.make_async_copy(tab_hbm.at[cen_v.at[pl.ds(0, NB * K)]],
                              e2_bufs[b], e2_sems[b]).wait()

    for b in range(NBUF):
        start_gather(b, b)

    def chunk_compute(j, b):
        wait_gather(b)
        e1b = e1_bufs[b]
        e2b = e2_bufs[b]
        accs = [jnp.zeros((16,), jnp.float32) for _ in range(YPC)]
        for r in range(NB):
            hacc = [e1b[r * C, pl.ds(d * 16, 16)] for d in range(HV)]
            for c in range(1, C):
                for d in range(HV):
                    hacc[d] = hacc[d] + e1b[r * C + c, pl.ds(d * 16, 16)]
            h = [a * (1.0 / C) for a in hacc]
            for k in range(K):
                q = r * K + k
                p0 = e2b[q, pl.ds(H, 16)] * h[0]
                p1 = e2b[q, pl.ds(H + 16, 16)] * h[1]
                p2 = e2b[q, pl.ds(H + 32, 16)] * h[2]
                p3 = e2b[q, pl.ds(H + 48, 16)] * h[3]
                s = jnp.sum((p0 + p1) + (p2 + p3))
                accs[q // 16] = jnp.where(masks[q % 16], s, accs[q // 16])
        ybase = j * (NB * K)
        for v in range(YPC):
            y = 1.0 / (1.0 + jnp.exp(-accs[v]))
            ybuf[pl.ds(pl.multiple_of(ybase + v * 16, 8), 16)] = y

    def loop_body(g, carry):
        for b in range(NBUF):
            j = g * NBUF + b
            chunk_compute(j, b)

            @pl.when(j + NBUF < NCHUNK)
            def _():
                start_gather(j + NBUF, b)
        return carry

    lax.fori_loop(0, NCHUNK // NBUF, loop_body, 0)

    out_base = pl.multiple_of(wid * (BPW * K), 8)
    pltpu.make_async_copy(ybuf, out_hbm.at[pl.ds(out_base, BPW * K)],
                          out_sem).start()
    pltpu.make_async_copy(ybuf, out_hbm.at[pl.ds(out_base, BPW * K)],
                          out_sem).wait()


@jax.jit
def _cbow_sc(ctx_flat, cen_flat, tab):
    mesh = plsc.VectorSubcoreMesh(core_axis_name="c", subcore_axis_name="s",
                                  num_cores=NC, num_subcores=NS)
    kern = pl.kernel(
        _cbow_body,
        out_type=jax.ShapeDtypeStruct((B * K,), jnp.float32),
        mesh=mesh,
        compiler_params=pltpu.CompilerParams(needs_layout_passes=False),
        scratch_types=[
            pltpu.VMEM((BPW * C,), jnp.int32),
            pltpu.VMEM((BPW * K,), jnp.int32),
            pltpu.VMEM((BPW * K,), jnp.float32),
            [pltpu.VMEM((NB * C, W), jnp.float32) for _ in range(NBUF)],
            [pltpu.VMEM((NB * K, W), jnp.float32) for _ in range(NBUF)],
            pltpu.SemaphoreType.DMA,
            [pltpu.SemaphoreType.DMA for _ in range(NBUF)],
            [pltpu.SemaphoreType.DMA for _ in range(NBUF)],
            pltpu.SemaphoreType.DMA,
        ],
    )
    return kern(ctx_flat, cen_flat, tab)


def kernel(contexts, centers, E1, E2):
    ctx_flat = contexts.reshape(B * C).astype(jnp.int32)
    cen_flat = centers.reshape(B * K).astype(jnp.int32)
    tab = jnp.concatenate([E1, E2], axis=1)
    y = _cbow_sc(ctx_flat, cen_flat, tab)
    return y.reshape(B, K)
